# Initial kernel scaffold; baseline (speedup 1.0000x reference)
import jax
import jax.numpy as jnp
from jax.experimental import pallas as pl


def kernel(input_token, table, gamma, beta):
    raise NotImplementedError("write your pallas kernel here")


# speedup vs baseline: 1.0170x; 1.0170x over previous
import functools

import jax
import jax.numpy as jnp
from jax import lax
from jax.experimental import pallas as pl
from jax.experimental.pallas import tpu as pltpu
from jax.experimental.pallas import tpu_sc as plsc

NC = 2
NS = 16
NW = NC * NS
EPS = 1e-5
NBUF = 3


def _ln_rows(rows, gvec, bvec, C, H):
    nv = H // 16
    g = [gvec[0, pl.ds(16 * j, 16)] for j in range(nv)]
    b = [bvec[0, pl.ds(16 * j, 16)] for j in range(nv)]
    inv_h = 1.0 / H

    @plsc.parallel_loop(0, C, 1, unroll=4)
    def body(r):
        x = [rows[r, pl.ds(16 * j, 16)] for j in range(nv)]
        s = (x[0] + x[1]) + (x[2] + x[3])
        q = (x[0] * x[0] + x[1] * x[1]) + (x[2] * x[2] + x[3] * x[3])
        mean = jnp.sum(s) * inv_h
        var = jnp.sum(q) * inv_h - mean * mean
        mean_v = jnp.full((16,), mean, dtype=jnp.float32)
        ve = jnp.full((16,), var + EPS, dtype=jnp.float32)
        i = lax.bitcast_convert_type(ve, jnp.int32)
        i = 0x5F3759DF - lax.shift_right_arithmetic(i, 1)
        y = lax.bitcast_convert_type(i, jnp.float32)
        xh = ve * 0.5
        y = y * (1.5 - xh * y * y)
        y = y * (1.5 - xh * y * y)
        y = y * (1.5 - xh * y * y)
        for j in range(nv):
            rows[r, pl.ds(16 * j, 16)] = (x[j] - mean_v) * y * g[j] + b[j]


@functools.partial(jax.jit, static_argnames=("n", "h"))
def _lookup_ln(idx, table, gamma, beta, n, h):
    npw = n // NW
    c = 400
    nchunk = npw // c

    mesh = plsc.VectorSubcoreMesh(
        core_axis_name="c", subcore_axis_name="s",
        num_cores=NC, num_subcores=NS)

    @functools.partial(
        pl.kernel,
        out_type=jax.ShapeDtypeStruct((n, h), jnp.float32),
        mesh=mesh,
        scratch_types=[
            pltpu.VMEM((npw,), jnp.int32),
            pltpu.VMEM((NBUF, c, h), jnp.float32),
            pltpu.VMEM((1, h), jnp.float32),
            pltpu.VMEM((1, h), jnp.float32),
            pltpu.SemaphoreType.DMA((NBUF,)),
            pltpu.SemaphoreType.DMA((NBUF,)),
        ],
        compiler_params=pltpu.CompilerParams(
            needs_layout_passes=False, use_tc_tiling_on_sc=False),
    )
    def k(idx_hbm, table_hbm, gamma_hbm, beta_hbm, out_hbm,
          idx_v, rows_v, g_v, b_v, gsem, osem):
        wid = lax.axis_index("s") * NC + lax.axis_index("c")
        base = wid * npw
        pltpu.sync_copy(idx_hbm.at[pl.ds(base, npw)], idx_v)
        pltpu.sync_copy(gamma_hbm, g_v.at[0])
        pltpu.sync_copy(beta_hbm, b_v.at[0])

        def gather(gk):
            return pltpu.async_copy(
                table_hbm.at[idx_v.at[pl.ds(gk * c, c)]],
                rows_v.at[gk % NBUF], gsem.at[gk % NBUF])

        def scatter(gk):
            return pltpu.async_copy(
                rows_v.at[gk % NBUF],
                out_hbm.at[pl.ds(base + gk * c, c)], osem.at[gk % NBUF])

        def wait_gather(gk):
            pltpu.make_async_copy(
                table_hbm.at[idx_v.at[pl.ds(gk * c, c)]],
                rows_v.at[gk % NBUF], gsem.at[gk % NBUF]).wait()

        def wait_scatter(gk):
            pltpu.make_async_copy(
                rows_v.at[gk % NBUF],
                out_hbm.at[pl.ds(base + gk * c, c)], osem.at[gk % NBUF]).wait()

        gather(0)
        if nchunk > 1:
            gather(1)
        for gk in range(nchunk):
            wait_gather(gk)
            _ln_rows(rows_v.at[gk % NBUF], g_v, b_v, c, h)
            scatter(gk)
            nxt = gk + 2
            if nxt < nchunk:
                if gk >= 1:
                    wait_scatter(gk - 1)
                gather(nxt)
        for gk in range(max(nchunk - NBUF, 0), nchunk):
            wait_scatter(gk)

    return k(idx, table, gamma, beta)


def kernel(input_token, table, gamma, beta):
    bsz, seq = input_token.shape
    vocab, h = table.shape
    n = bsz * seq
    idx = input_token.reshape(n)
    out = _lookup_ln(idx, table, gamma, beta, n, h)
    return out.reshape(bsz, seq, h)
